# Initial kernel scaffold; baseline (speedup 1.0000x reference)
import jax
import jax.numpy as jnp
from jax.experimental import pallas as pl


def kernel(x, pos_embed):
    raise NotImplementedError("write your pallas kernel here")


# speedup vs baseline: 1.2637x; 1.2637x over previous
import jax
import jax.numpy as jnp
from jax.experimental import pallas as pl


def _add_pos_kernel(x_ref, pos_ref, o_ref):
    h = x_ref.shape[1]
    w = x_ref.shape[2]
    o_ref[...] = x_ref[...] + pos_ref[:h, :w, :][None]


def kernel(x, pos_embed):
    B, H, W, C = x.shape
    out = pl.pallas_call(
        _add_pos_kernel,
        grid=(B,),
        in_specs=[
            pl.BlockSpec((1, H, W, C), lambda b: (b, 0, 0, 0)),
            pl.BlockSpec(pos_embed.shape, lambda b: (0, 0, 0)),
        ],
        out_specs=pl.BlockSpec((1, H, W, C), lambda b: (b, 0, 0, 0)),
        out_shape=jax.ShapeDtypeStruct(x.shape, x.dtype),
    )(x, pos_embed)
    return out


# speedup vs baseline: 1.2960x; 1.0256x over previous
import jax
import jax.numpy as jnp
from jax.experimental import pallas as pl


def _add_pos_kernel(x_ref, pos_ref, o_ref):
    h = x_ref.shape[1]
    w = x_ref.shape[2]
    o_ref[...] = x_ref[...] + pos_ref[:h, :w, :][None]


def kernel(x, pos_embed):
    B, H, W, C = x.shape
    out = pl.pallas_call(
        _add_pos_kernel,
        grid=(B,),
        in_specs=[
            pl.BlockSpec((1, H, W, C), lambda b: (b, 0, 0, 0)),
            pl.BlockSpec((H, pos_embed.shape[1], C), lambda b: (0, 0, 0)),
        ],
        out_specs=pl.BlockSpec((1, H, W, C), lambda b: (b, 0, 0, 0)),
        out_shape=jax.ShapeDtypeStruct(x.shape, x.dtype),
    )(x, pos_embed)
    return out


# speedup vs baseline: 1.3172x; 1.0164x over previous
import jax
import jax.numpy as jnp
from jax.experimental import pallas as pl


def _add_pos_kernel(x_ref, pos_ref, o_ref):
    h = x_ref.shape[1]
    w = x_ref.shape[2]
    o_ref[...] = x_ref[...] + pos_ref[:h, :w, :][None]


def kernel(x, pos_embed):
    B, H, W, C = x.shape
    out = pl.pallas_call(
        _add_pos_kernel,
        grid=(B,),
        in_specs=[
            pl.BlockSpec((1, H, W, C), lambda b: (b, 0, 0, 0)),
            pl.BlockSpec((H, W, C), lambda b: (0, 0, 0)),
        ],
        out_specs=pl.BlockSpec((1, H, W, C), lambda b: (b, 0, 0, 0)),
        out_shape=jax.ShapeDtypeStruct(x.shape, x.dtype),
    )(x, pos_embed)
    return out
